# baseline (speedup 1.0000x reference)
import functools

import jax
import jax.numpy as jnp
from jax import lax
from jax.experimental import pallas as pl
from jax.experimental.pallas import tpu as pltpu
from jax.experimental.pallas import tpu_sc as plsc

B = 16384
K = 64
NC = 2
NS = 16
NW = NC * NS
BPW = B // NW
CHUNK = 128
NCHUNK = BPW // CHUNK
GROUPS = CHUNK // 16

_mesh = plsc.VectorSubcoreMesh(core_axis_name="c", subcore_axis_name="s")


@functools.partial(
    pl.kernel,
    out_type=jax.ShapeDtypeStruct((NW, NCHUNK, CHUNK), jnp.float32),
    mesh=_mesh,
    compiler_params=pltpu.CompilerParams(use_tc_tiling_on_sc=False),
    scratch_types=[
        pltpu.VMEM((NCHUNK, CHUNK), jnp.int32),
        pltpu.VMEM((NCHUNK, CHUNK), jnp.int32),
        pltpu.VMEM((K, NCHUNK, CHUNK), jnp.float32),
        pltpu.VMEM((K, NCHUNK, CHUNK), jnp.float32),
        pltpu.VMEM((NCHUNK, CHUNK), jnp.float32),
        pltpu.VMEM((NCHUNK, CHUNK), jnp.float32),
        pltpu.VMEM((NCHUNK, CHUNK), jnp.float32),
        pltpu.SemaphoreType.DMA,
    ],
)
def _mf_sc(user_hbm, item_hbm, uet_hbm, iet_hbm, ub_hbm, ib_hbm, out_hbm,
           raw_u, raw_i, val_u, val_i, bias_u, bias_i, out_v, sem):
    wid = lax.axis_index("s") * NC + lax.axis_index("c")

    pltpu.sync_copy(user_hbm.at[wid], raw_u)
    pltpu.sync_copy(item_hbm.at[wid], raw_i)

    n_fired = 0
    for c in range(NCHUNK):
        pltpu.async_copy(ub_hbm.at[raw_u.at[c]], bias_u.at[c], sem)
        pltpu.async_copy(ib_hbm.at[raw_i.at[c]], bias_i.at[c], sem)
        n_fired += 2

    for k in range(K):
        for c in range(NCHUNK):
            pltpu.async_copy(uet_hbm.at[k].at[raw_u.at[c]], val_u.at[k, c], sem)
            pltpu.async_copy(iet_hbm.at[k].at[raw_i.at[c]], val_i.at[k, c], sem)
            n_fired += 2

    def drain_body(i, _):
        pltpu.make_async_copy(ub_hbm.at[pl.ds(0, CHUNK)], out_v.at[0], sem).wait()
        return _

    lax.fori_loop(0, n_fired, drain_body, 0)

    for c in range(NCHUNK):
        def g_body(g, _, c=c):
            sl = pl.ds(g * 16, 16)
            acc = bias_u[c, sl] + bias_i[c, sl]
            for k in range(K):
                acc = acc + val_u[k, c, sl] * val_i[k, c, sl]
            out_v[c, sl] = acc
            return _
        lax.fori_loop(0, GROUPS, g_body, 0)

    pltpu.sync_copy(out_v, out_hbm.at[wid])


def kernel(user, item, user_embedding, item_embedding, user_bias, item_bias):
    user = user.astype(jnp.int32).reshape(NW, NCHUNK, CHUNK)
    item = item.astype(jnp.int32).reshape(NW, NCHUNK, CHUNK)
    uet = user_embedding.T
    iet = item_embedding.T
    ub = user_bias.reshape(-1)
    ib = item_bias.reshape(-1)
    out = _mf_sc(user, item, uet, iet, ub, ib)
    return out.reshape(B)
